# baseline (speedup 1.0000x reference)
import functools

import jax
import jax.numpy as jnp
import numpy as np
from jax import lax
from jax.experimental import pallas as pl
from jax.experimental.pallas import tpu as pltpu
from jax.experimental.pallas import tpu_sc as plsc

_B = 256
_D = 16384
_P = 256
_KF = 60
_KC = 40
_KFP = 64
_KCP = 48
_REG = 0.05
_ITERS = 20
_ALPHA = 0.5
_LAM_F = 1.2
_LAM_C = 0.5
_RBLOCK = 4096
_NEG = -1e30
_BIG = 1e30

_DB = 2048


def _tf_rounds(x0, x1, k1, k2):
    ks0, ks1 = np.uint32(k1), np.uint32(k2)
    ks2 = ks0 ^ ks1 ^ np.uint32(0x1BD11BDA)
    rot0, rot1 = (13, 15, 26, 6), (17, 29, 16, 24)
    x0 = x0 + ks0
    x1 = x1 + ks1

    def rnds(x0, x1, rots):
        for r in rots:
            r32, c32 = np.uint32(r), np.uint32(32 - r)
            x0 = x0 + x1
            x1 = (x1 << r32) | (x1 >> c32)
            x1 = x1 ^ x0
        return x0, x1

    ks = (ks0, ks1, ks2)
    for i in range(5):
        x0, x1 = rnds(x0, x1, rot0 if i % 2 == 0 else rot1)
        x0 = x0 + ks[(i + 1) % 3]
        x1 = x1 + ks[(i + 2) % 3] + np.uint32(i + 1)
    return x0, x1


def _rand01(seed: int, n: int) -> np.ndarray:
    k1 = np.uint32((seed >> 32) & 0xFFFFFFFF)
    k2 = np.uint32(seed & 0xFFFFFFFF)
    b1, b2 = _tf_rounds(np.zeros(2, np.uint32),
                        np.arange(2, dtype=np.uint32), k1, k2)
    o1, o2 = _tf_rounds(np.zeros(n, np.uint32),
                        np.arange(n, dtype=np.uint32), b1[1], b2[1])
    return ((o1 ^ o2) & np.uint32(1)).astype(np.int8)


def _pi_tables() -> np.ndarray:
    tabs = []
    for seed in (10, 20, 30, 11, 21, 31):
        blocks = [
            _rand01(seed + start, _RBLOCK * _P).reshape(_RBLOCK, _P) * 2 - 1
            for start in range(0, _D, _RBLOCK)
        ]
        tabs.append(np.concatenate(blocks, axis=0))
    return np.stack(tabs)


_PI = _pi_tables()


def _proj_body(g0, g1, g2, g3, g4, g5, pi_ref, out_ref):
    k = pl.program_id(0)
    for h, g in enumerate((g0, g1, g2, g3, g4, g5)):
        acc = jnp.dot(g[...], pi_ref[h].astype(jnp.float32),
                      preferred_element_type=jnp.float32)

        @pl.when(k == 0)
        def _(acc=acc, h=h):
            out_ref[h] = acc

        @pl.when(k != 0)
        def _(acc=acc, h=h):
            out_ref[h] = out_ref[h] + acc


def _project(gs):
    pi = jnp.asarray(_PI)
    g_spec = pl.BlockSpec((_B, _DB), lambda k: (0, k))
    return pl.pallas_call(
        _proj_body,
        grid=(_D // _DB,),
        in_specs=[g_spec] * 6 + [pl.BlockSpec((6, _DB, _P), lambda k: (0, k, 0))],
        out_specs=pl.BlockSpec((6, _B, _P), lambda k: (0, 0, 0)),
        out_shape=jax.ShapeDtypeStruct((6, _B, _P), jnp.float32),
        compiler_params=pltpu.CompilerParams(
            dimension_semantics=("arbitrary",)),
    )(*gs, pi)


def _nrm(x):
    n = jnp.maximum(jnp.sqrt(jnp.sum(x * x, axis=1, keepdims=True)), 1e-12)
    return x / n


def _costs_body(p_ref, cf_ref, cv_ref, ct_ref, wv_ref, wt_ref, wf_ref,
                costf_ref, costc_ref, cfn_ref, cvn_ref, ctn_ref,
                gsf_ref, gsv_ref, gst_ref):
    gTv = _nrm(p_ref[0])
    gTt = _nrm(p_ref[1])
    gTf = _nrm(p_ref[2])
    gSv = _nrm(p_ref[3])
    gSt = _nrm(p_ref[4])
    gSf = _nrm(p_ref[5])

    def mat_t(a, w):
        return lax.dot_general(a, w, (((1,), (1,)), ((), ())),
                               preferred_element_type=jnp.float32)

    gsv_ref[...] = mat_t(gSv, wv_ref[...])
    gst_ref[...] = mat_t(gSt, wt_ref[...])
    gsf_ref[...] = mat_t(gSf, wf_ref[...])

    Cf = _nrm(cf_ref[...])
    Cv = _nrm(cv_ref[...])
    Ct = _nrm(ct_ref[...])
    cfn_ref[...] = Cf
    cvn_ref[...] = Cv
    ctn_ref[...] = Ct

    def sqe(a, b):
        a2 = jnp.sum(a * a, axis=1, keepdims=True)
        b2 = jnp.sum(b * b, axis=1)[None, :]
        ab = mat_t(a, b)
        return jnp.maximum(a2 + b2 - 2.0 * ab, 0.0)

    cost_f = sqe(gTf, Cf)
    cost_c = _ALPHA * sqe(gTv, Cv) + (1.0 - _ALPHA) * sqe(gTt, Ct)

    mf = lax.broadcasted_iota(jnp.int32, (1, _KFP), 1) < _KF
    mc = lax.broadcasted_iota(jnp.int32, (1, _KCP), 1) < _KC
    costf_ref[...] = jnp.where(mf, cost_f, _BIG)
    costc_ref[...] = jnp.where(mc, cost_c, _BIG)


def _costs(p, cf, cv, ct, wv, wt, wf):
    f32 = jnp.float32
    return pl.pallas_call(
        _costs_body,
        out_shape=(
            jax.ShapeDtypeStruct((_B, _KFP), f32),
            jax.ShapeDtypeStruct((_B, _KCP), f32),
            jax.ShapeDtypeStruct((_KFP, _P), f32),
            jax.ShapeDtypeStruct((_KCP, _P), f32),
            jax.ShapeDtypeStruct((_KCP, _P), f32),
            jax.ShapeDtypeStruct((_B, _P), f32),
            jax.ShapeDtypeStruct((_B, _P), f32),
            jax.ShapeDtypeStruct((_B, _P), f32),
        ),
    )(p, cf, cv, ct, wv, wt, wf)


def _sinkhorn_body(costf_ref, costc_ref, out_ref):
    cost_f = costf_ref[...]
    cost_c = costc_ref[...]
    mf = lax.broadcasted_iota(jnp.int32, (1, _KFP), 1) < _KF
    mc = lax.broadcasted_iota(jnp.int32, (1, _KCP), 1) < _KC

    Kf = jnp.exp(cost_f * (-1.0 / _REG))
    Kc = jnp.exp(cost_c * (-1.0 / _REG))
    KfT = jnp.transpose(Kf)
    KcT = jnp.transpose(Kc)

    af = jnp.float32(1.0 / _B)
    bf = jnp.float32(1.0 / _KF)
    bc = jnp.float32(1.0 / _KC)

    def mm(a, w):
        return lax.dot_general(a, w, (((1,), (0,)), ((), ())),
                               preferred_element_type=jnp.float32)

    def one_iter(carry):
        uf, vf, uc, vc = carry
        uf = af / mm(vf, KfT)
        vf = jnp.where(mf, bf / mm(uf, Kf), 0.0)
        uc = af / mm(vc, KcT)
        vc = jnp.where(mc, bc / mm(uc, Kc), 0.0)
        return uf, vf, uc, vc

    _UNROLL = 5

    def step(_, carry):
        for _ in range(_UNROLL):
            carry = one_iter(carry)
        return carry

    init = (jnp.full((1, _B), af), jnp.where(mf, 1.0, 0.0).astype(jnp.float32),
            jnp.full((1, _B), af), jnp.where(mc, 1.0, 0.0).astype(jnp.float32))
    uf, vf, uc, vc = lax.fori_loop(0, _ITERS // _UNROLL, step, init)

    out_ref[0, 0] = jnp.sum(mm(uf, Kf * cost_f) * vf)
    out_ref[0, 1] = jnp.sum(mm(uc, Kc * cost_c) * vc)


def _sinkhorn(costf, costc):
    return pl.pallas_call(
        _sinkhorn_body,
        out_shape=jax.ShapeDtypeStruct((1, 2), jnp.float32),
        out_specs=pl.BlockSpec(memory_space=pltpu.SMEM),
    )(costf, costc)


_NC, _NS = 2, 16
_NW = _NC * _NS
_RPW = _B // _NW


def _sc_vq_body(costf_hbm, costc_hbm, cf_hbm, cv_hbm, ct_hbm,
                gsf_hbm, gsv_hbm, gst_hbm, out_hbm,
                costf_v, costc_v, idxf_v, idxc_v,
                rowf_v, rowv_v, rowt_v, gf_v, gv_v, gt_v, acc_v, sem):
    wid = lax.axis_index("s") * _NC + lax.axis_index("c")
    base = wid * _RPW
    lanes = lax.iota(jnp.int32, 16)

    pltpu.sync_copy(costf_hbm.at[pl.ds(base, _RPW)], costf_v)
    pltpu.sync_copy(costc_hbm.at[pl.ds(base, _RPW)], costc_v)
    pltpu.sync_copy(gsf_hbm.at[pl.ds(base, _RPW)], gf_v)
    pltpu.sync_copy(gsv_hbm.at[pl.ds(base, _RPW)], gv_v)
    pltpu.sync_copy(gst_hbm.at[pl.ds(base, _RPW)], gt_v)

    _dnums = lax.GatherDimensionNumbers(
        offset_dims=(), collapsed_slice_dims=(0,), start_index_map=(0,))

    def lperm(x, s):
        idx = jnp.reshape(lanes ^ s, (16, 1))
        return lax.gather(x, idx, _dnums, slice_sizes=(1,),
                          mode=lax.GatherScatterMode.PROMISE_IN_BOUNDS)

    def xmin(x):
        for s in (8, 4, 2, 1):
            x = jnp.minimum(x, lperm(x, s))
        return x

    def row_argmin(cost_v, kp, r):
        vmin = cost_v[r, pl.ds(0, 16)]
        vidx = lanes
        for j in range(1, kp // 16):
            x = cost_v[r, pl.ds(16 * j, 16)]
            lt = x < vmin
            vidx = jnp.where(lt, lanes + 16 * j, vidx)
            vmin = jnp.where(lt, x, vmin)
        gmin = xmin(vmin)
        return xmin(jnp.where(vmin == gmin, vidx, kp))

    kf = jnp.zeros((16,), jnp.int32)
    kc = jnp.zeros((16,), jnp.int32)
    for r in range(_RPW):
        kf = jnp.where(lanes == r, row_argmin(costf_v, _KFP, r), kf)
        kc = jnp.where(lanes == r, row_argmin(costc_v, _KCP, r), kc)
    idxf_v[...] = kf
    idxc_v[...] = kc

    pltpu.async_copy(cf_hbm.at[idxf_v], rowf_v, sem).wait()
    pltpu.async_copy(cv_hbm.at[idxc_v], rowv_v, sem).wait()
    pltpu.async_copy(ct_hbm.at[idxc_v], rowt_v, sem).wait()

    for i, (g_v, row_v) in enumerate(((gf_v, rowf_v), (gv_v, rowv_v),
                                      (gt_v, rowt_v))):
        acc = jnp.zeros((16,), jnp.float32)
        for r in range(_RPW):
            for j in range(_P // 16):
                d = g_v[r, pl.ds(16 * j, 16)] - row_v[r, pl.ds(16 * j, 16)]
                acc = acc + d * d
        acc_v[i] = acc

    pltpu.sync_copy(acc_v, out_hbm.at[wid])


@functools.partial(
    pl.kernel,
    out_type=jax.ShapeDtypeStruct((_NW, 3, 16), jnp.float32),
    mesh=plsc.VectorSubcoreMesh(core_axis_name="c", subcore_axis_name="s"),
    scratch_types=[
        pltpu.VMEM((_RPW, _KFP), jnp.float32),
        pltpu.VMEM((_RPW, _KCP), jnp.float32),
        pltpu.VMEM((16,), jnp.int32),
        pltpu.VMEM((16,), jnp.int32),
        pltpu.VMEM((16, _P), jnp.float32),
        pltpu.VMEM((16, _P), jnp.float32),
        pltpu.VMEM((16, _P), jnp.float32),
        pltpu.VMEM((_RPW, _P), jnp.float32),
        pltpu.VMEM((_RPW, _P), jnp.float32),
        pltpu.VMEM((_RPW, _P), jnp.float32),
        pltpu.VMEM((3, 16), jnp.float32),
        pltpu.SemaphoreType.DMA,
    ],
)
def _sc_vq(*refs):
    _sc_vq_body(*refs)


def kernel(g_t_v, g_t_t, g_t_f, g_s_v, g_s_t, g_s_f, fusion_centroids,
           v_centroids, t_centroids, W_v, W_t, W_f):
    p = _project((g_t_v, g_t_t, g_t_f, g_s_v, g_s_t, g_s_f))
    cf = jnp.pad(fusion_centroids, ((0, _KFP - _KF), (0, 0)))
    cv = jnp.pad(v_centroids, ((0, _KCP - _KC), (0, 0)))
    ct = jnp.pad(t_centroids, ((0, _KCP - _KC), (0, 0)))
    costf, costc, cfn, cvn, ctn, gsf, gsv, gst = _costs(
        p, cf, cv, ct, W_v, W_t, W_f)
    ots = _sinkhorn(costf, costc)
    parts = _sc_vq(costf, costc, cfn, cvn, ctn, gsf, gsv, gst)
    s = jnp.sum(parts, axis=(0, 2))
    return (_LAM_F * (ots[0, 0] + s[0])
            + _LAM_C * (ots[0, 1] + _ALPHA * s[1] + (1.0 - _ALPHA) * s[2]))


# speedup vs baseline: 1.0902x; 1.0902x over previous
import functools

import jax
import jax.numpy as jnp
import numpy as np
from jax import lax
from jax.experimental import pallas as pl
from jax.experimental.pallas import tpu as pltpu
from jax.experimental.pallas import tpu_sc as plsc

_B = 256
_D = 16384
_P = 256
_KF = 60
_KC = 40
_KFP = 64
_KCP = 48
_REG = 0.05
_ITERS = 20
_ALPHA = 0.5
_LAM_F = 1.2
_LAM_C = 0.5
_RBLOCK = 4096
_NEG = -1e30
_BIG = 1e30

_DB = 2048


def _tf_rounds(x0, x1, k1, k2):
    ks0, ks1 = np.uint32(k1), np.uint32(k2)
    ks2 = ks0 ^ ks1 ^ np.uint32(0x1BD11BDA)
    rot0, rot1 = (13, 15, 26, 6), (17, 29, 16, 24)
    x0 = x0 + ks0
    x1 = x1 + ks1

    def rnds(x0, x1, rots):
        for r in rots:
            r32, c32 = np.uint32(r), np.uint32(32 - r)
            x0 = x0 + x1
            x1 = (x1 << r32) | (x1 >> c32)
            x1 = x1 ^ x0
        return x0, x1

    ks = (ks0, ks1, ks2)
    for i in range(5):
        x0, x1 = rnds(x0, x1, rot0 if i % 2 == 0 else rot1)
        x0 = x0 + ks[(i + 1) % 3]
        x1 = x1 + ks[(i + 2) % 3] + np.uint32(i + 1)
    return x0, x1


def _rand01(seed: int, n: int) -> np.ndarray:
    k1 = np.uint32((seed >> 32) & 0xFFFFFFFF)
    k2 = np.uint32(seed & 0xFFFFFFFF)
    b1, b2 = _tf_rounds(np.zeros(2, np.uint32),
                        np.arange(2, dtype=np.uint32), k1, k2)
    o1, o2 = _tf_rounds(np.zeros(n, np.uint32),
                        np.arange(n, dtype=np.uint32), b1[1], b2[1])
    return ((o1 ^ o2) & np.uint32(1)).astype(np.int8)


def _pi_tables() -> np.ndarray:
    tabs = []
    for seed in (10, 20, 30, 11, 21, 31):
        blocks = [
            _rand01(seed + start, _RBLOCK * _P).reshape(_RBLOCK, _P) * 2 - 1
            for start in range(0, _D, _RBLOCK)
        ]
        tabs.append(np.concatenate(blocks, axis=0))
    return np.stack(tabs)


_PI = _pi_tables()


def _proj_body(g0, g1, g2, g3, g4, g5, pi_ref, out_ref):
    k = pl.program_id(0)
    for h, g in enumerate((g0, g1, g2, g3, g4, g5)):
        acc = jnp.dot(g[...], pi_ref[h].astype(jnp.float32),
                      preferred_element_type=jnp.float32)

        @pl.when(k == 0)
        def _(acc=acc, h=h):
            out_ref[h] = acc

        @pl.when(k != 0)
        def _(acc=acc, h=h):
            out_ref[h] = out_ref[h] + acc


def _project(gs):
    pi = jnp.asarray(_PI)
    g_spec = pl.BlockSpec((_B, _DB), lambda k: (0, k))
    return pl.pallas_call(
        _proj_body,
        grid=(_D // _DB,),
        in_specs=[g_spec] * 6 + [pl.BlockSpec((6, _DB, _P), lambda k: (0, k, 0))],
        out_specs=pl.BlockSpec((6, _B, _P), lambda k: (0, 0, 0)),
        out_shape=jax.ShapeDtypeStruct((6, _B, _P), jnp.float32),
        compiler_params=pltpu.CompilerParams(
            dimension_semantics=("arbitrary",)),
    )(*gs, pi)


def _nrm(x):
    n = jnp.maximum(jnp.sqrt(jnp.sum(x * x, axis=1, keepdims=True)), 1e-12)
    return x / n


def _costs_body(p_ref, cf_ref, cv_ref, ct_ref, wv_ref, wt_ref, wf_ref,
                costf_ref, costc_ref, cfn_ref, cvn_ref, ctn_ref,
                gsf_ref, gsv_ref, gst_ref):
    gTv = _nrm(p_ref[0])
    gTt = _nrm(p_ref[1])
    gTf = _nrm(p_ref[2])
    gSv = _nrm(p_ref[3])
    gSt = _nrm(p_ref[4])
    gSf = _nrm(p_ref[5])

    def mat_t(a, w):
        return lax.dot_general(a, w, (((1,), (1,)), ((), ())),
                               preferred_element_type=jnp.float32)

    gsv_ref[...] = mat_t(gSv, wv_ref[...])
    gst_ref[...] = mat_t(gSt, wt_ref[...])
    gsf_ref[...] = mat_t(gSf, wf_ref[...])

    Cf = _nrm(cf_ref[...])
    Cv = _nrm(cv_ref[...])
    Ct = _nrm(ct_ref[...])
    cfn_ref[...] = Cf
    cvn_ref[...] = Cv
    ctn_ref[...] = Ct

    def sqe(a, b):
        a2 = jnp.sum(a * a, axis=1, keepdims=True)
        b2 = jnp.sum(b * b, axis=1)[None, :]
        ab = mat_t(a, b)
        return jnp.maximum(a2 + b2 - 2.0 * ab, 0.0)

    cost_f = sqe(gTf, Cf)
    cost_c = _ALPHA * sqe(gTv, Cv) + (1.0 - _ALPHA) * sqe(gTt, Ct)

    mf = lax.broadcasted_iota(jnp.int32, (1, _KFP), 1) < _KF
    mc = lax.broadcasted_iota(jnp.int32, (1, _KCP), 1) < _KC
    costf_ref[...] = jnp.where(mf, cost_f, _BIG)
    costc_ref[...] = jnp.where(mc, cost_c, _BIG)


def _costs(p, cf, cv, ct, wv, wt, wf):
    f32 = jnp.float32
    return pl.pallas_call(
        _costs_body,
        out_shape=(
            jax.ShapeDtypeStruct((_B, _KFP), f32),
            jax.ShapeDtypeStruct((_B, _KCP), f32),
            jax.ShapeDtypeStruct((_KFP, _P), f32),
            jax.ShapeDtypeStruct((_KCP, _P), f32),
            jax.ShapeDtypeStruct((_KCP, _P), f32),
            jax.ShapeDtypeStruct((_B, _P), f32),
            jax.ShapeDtypeStruct((_B, _P), f32),
            jax.ShapeDtypeStruct((_B, _P), f32),
        ),
    )(p, cf, cv, ct, wv, wt, wf)


def _sinkhorn_body(costf_ref, costc_ref, out_ref):
    cost_f = costf_ref[...]
    cost_c = costc_ref[...]
    mf = lax.broadcasted_iota(jnp.int32, (1, _KFP), 1) < _KF
    mc = lax.broadcasted_iota(jnp.int32, (1, _KCP), 1) < _KC

    Kf = jnp.exp(cost_f * (-1.0 / _REG))
    Kc = jnp.exp(cost_c * (-1.0 / _REG))
    KfT = jnp.transpose(Kf)
    KcT = jnp.transpose(Kc)

    af = jnp.float32(1.0 / _B)
    bf = jnp.float32(1.0 / _KF)
    bc = jnp.float32(1.0 / _KC)

    def mm(a, w):
        return lax.dot_general(a, w, (((1,), (0,)), ((), ())),
                               preferred_element_type=jnp.float32)

    def one_iter(carry):
        uf, vf, uc, vc = carry
        uf = af / mm(vf, KfT)
        vf = jnp.where(mf, bf / mm(uf, Kf), 0.0)
        uc = af / mm(vc, KcT)
        vc = jnp.where(mc, bc / mm(uc, Kc), 0.0)
        return uf, vf, uc, vc

    _UNROLL = 5

    def step(_, carry):
        for _ in range(_UNROLL):
            carry = one_iter(carry)
        return carry

    init = (jnp.full((1, _B), af), jnp.where(mf, 1.0, 0.0).astype(jnp.float32),
            jnp.full((1, _B), af), jnp.where(mc, 1.0, 0.0).astype(jnp.float32))
    uf, vf, uc, vc = lax.fori_loop(0, _ITERS // _UNROLL, step, init)

    out_ref[0, 0] = jnp.sum(mm(uf, Kf * cost_f) * vf)
    out_ref[0, 1] = jnp.sum(mm(uc, Kc * cost_c) * vc)


def _sinkhorn(costf, costc):
    return pl.pallas_call(
        _sinkhorn_body,
        out_shape=jax.ShapeDtypeStruct((1, 2), jnp.float32),
        out_specs=pl.BlockSpec(memory_space=pltpu.SMEM),
    )(costf, costc)


_NC, _NS = 2, 16
_NW = _NC * _NS
_RPW = _B // _NW


def _sc_vq_body(costf_hbm, costc_hbm, cf_hbm, cv_hbm, ct_hbm,
                gsf_hbm, gsv_hbm, gst_hbm, out_hbm,
                costf_v, costc_v, idxf_v, idxc_v,
                rowf_v, rowv_v, rowt_v, gf_v, gv_v, gt_v, acc_v, sem, sem2):
    wid = lax.axis_index("s") * _NC + lax.axis_index("c")
    base = wid * _RPW
    lanes = lax.iota(jnp.int32, 16)

    c1 = pltpu.async_copy(costf_hbm.at[pl.ds(base, _RPW)], costf_v, sem)
    c2 = pltpu.async_copy(costc_hbm.at[pl.ds(base, _RPW)], costc_v, sem)
    g1 = pltpu.async_copy(gsf_hbm.at[pl.ds(base, _RPW)], gf_v, sem2)
    g2 = pltpu.async_copy(gsv_hbm.at[pl.ds(base, _RPW)], gv_v, sem2)
    g3 = pltpu.async_copy(gst_hbm.at[pl.ds(base, _RPW)], gt_v, sem2)
    c1.wait()
    c2.wait()

    _dnums = lax.GatherDimensionNumbers(
        offset_dims=(), collapsed_slice_dims=(0,), start_index_map=(0,))

    def lperm(x, s):
        idx = jnp.reshape(lanes ^ s, (16, 1))
        return lax.gather(x, idx, _dnums, slice_sizes=(1,),
                          mode=lax.GatherScatterMode.PROMISE_IN_BOUNDS)

    def xmin(x):
        for s in (8, 4, 2, 1):
            x = jnp.minimum(x, lperm(x, s))
        return x

    def row_argmin(cost_v, kp, r):
        vmin = cost_v[r, pl.ds(0, 16)]
        vidx = lanes
        for j in range(1, kp // 16):
            x = cost_v[r, pl.ds(16 * j, 16)]
            lt = x < vmin
            vidx = jnp.where(lt, lanes + 16 * j, vidx)
            vmin = jnp.where(lt, x, vmin)
        gmin = xmin(vmin)
        return xmin(jnp.where(vmin == gmin, vidx, kp))

    kf = jnp.zeros((16,), jnp.int32)
    kc = jnp.zeros((16,), jnp.int32)
    for r in range(_RPW):
        kf = jnp.where(lanes == r, row_argmin(costf_v, _KFP, r), kf)
        kc = jnp.where(lanes == r, row_argmin(costc_v, _KCP, r), kc)
    idxf_v[...] = kf
    idxc_v[...] = kc

    r1 = pltpu.async_copy(cf_hbm.at[idxf_v], rowf_v, sem)
    r2 = pltpu.async_copy(cv_hbm.at[idxc_v], rowv_v, sem)
    r3 = pltpu.async_copy(ct_hbm.at[idxc_v], rowt_v, sem)
    g1.wait()
    g2.wait()
    g3.wait()
    r1.wait()
    r2.wait()
    r3.wait()

    for i, (g_v, row_v) in enumerate(((gf_v, rowf_v), (gv_v, rowv_v),
                                      (gt_v, rowt_v))):
        acc = jnp.zeros((16,), jnp.float32)
        for r in range(_RPW):
            for j in range(_P // 16):
                d = g_v[r, pl.ds(16 * j, 16)] - row_v[r, pl.ds(16 * j, 16)]
                acc = acc + d * d
        acc_v[i] = acc

    pltpu.sync_copy(acc_v, out_hbm.at[wid])


@functools.partial(
    pl.kernel,
    out_type=jax.ShapeDtypeStruct((_NW, 3, 16), jnp.float32),
    mesh=plsc.VectorSubcoreMesh(core_axis_name="c", subcore_axis_name="s"),
    scratch_types=[
        pltpu.VMEM((_RPW, _KFP), jnp.float32),
        pltpu.VMEM((_RPW, _KCP), jnp.float32),
        pltpu.VMEM((16,), jnp.int32),
        pltpu.VMEM((16,), jnp.int32),
        pltpu.VMEM((16, _P), jnp.float32),
        pltpu.VMEM((16, _P), jnp.float32),
        pltpu.VMEM((16, _P), jnp.float32),
        pltpu.VMEM((_RPW, _P), jnp.float32),
        pltpu.VMEM((_RPW, _P), jnp.float32),
        pltpu.VMEM((_RPW, _P), jnp.float32),
        pltpu.VMEM((3, 16), jnp.float32),
        pltpu.SemaphoreType.DMA,
        pltpu.SemaphoreType.DMA,
    ],
)
def _sc_vq(*refs):
    _sc_vq_body(*refs)


def kernel(g_t_v, g_t_t, g_t_f, g_s_v, g_s_t, g_s_f, fusion_centroids,
           v_centroids, t_centroids, W_v, W_t, W_f):
    p = _project((g_t_v, g_t_t, g_t_f, g_s_v, g_s_t, g_s_f))
    cf = jnp.pad(fusion_centroids, ((0, _KFP - _KF), (0, 0)))
    cv = jnp.pad(v_centroids, ((0, _KCP - _KC), (0, 0)))
    ct = jnp.pad(t_centroids, ((0, _KCP - _KC), (0, 0)))
    costf, costc, cfn, cvn, ctn, gsf, gsv, gst = _costs(
        p, cf, cv, ct, W_v, W_t, W_f)
    ots = _sinkhorn(costf, costc)
    parts = _sc_vq(costf, costc, cfn, cvn, ctn, gsf, gsv, gst)
    s = jnp.sum(parts, axis=(0, 2))
    return (_LAM_F * (ots[0, 0] + s[0])
            + _LAM_C * (ots[0, 1] + _ALPHA * s[1] + (1.0 - _ALPHA) * s[2]))
